# Initial kernel scaffold; baseline (speedup 1.0000x reference)
import jax
import jax.numpy as jnp
from jax.experimental import pallas as pl


def kernel(x, table):
    raise NotImplementedError("write your pallas kernel here")


# speedup vs baseline: 4.0660x; 4.0660x over previous
import functools

import jax
import jax.numpy as jnp
from jax import lax
from jax.experimental import pallas as pl
from jax.experimental.pallas import tpu as pltpu
from jax.experimental.pallas import tpu_sc as plsc

MAXLEN = 8192
DM = 768
BATCH = 4
HALF = MAXLEN // 2
NC = 2
NS = 16
NW = NC * NS
K = HALF // NW
L = 16

_mesh = plsc.VectorSubcoreMesh(core_axis_name="c", subcore_axis_name="s")


@functools.partial(
    pl.kernel,
    mesh=_mesh,
    out_type=jax.ShapeDtypeStruct((BATCH * MAXLEN, DM), jnp.float32),
    scratch_types=[
        pltpu.VMEM((K, DM), jnp.float32),
        pltpu.VMEM((BATCH, K), jnp.int32),
        pltpu.VMEM((1, DM), jnp.float32),
        pltpu.SemaphoreType.DMA,
    ],
)
def _emb(table_hbm, out_hbm, rows_v, idx_v, row0_v, sem):
    wid = lax.axis_index("s") * NC + lax.axis_index("c")
    s = wid * K

    pltpu.sync_copy(table_hbm.at[pl.ds(s, K)], rows_v)

    lane = lax.iota(jnp.int32, L)
    for b in range(BATCH):
        base = (b * MAXLEN + HALF) - s
        for j in range(K // L):
            idx_v[b, pl.ds(j * L, L)] = (base - j * L) - lane

    copies = []
    for b in range(BATCH):
        copies.append(
            pltpu.async_copy(
                rows_v, out_hbm.at[pl.ds(b * MAXLEN + HALF + s, K)], sem
            )
        )
        copies.append(pltpu.async_copy(rows_v, out_hbm.at[idx_v.at[b]], sem))
    for c in copies:
        c.wait()

    @pl.when(wid == 0)
    def _():
        pltpu.sync_copy(table_hbm.at[pl.ds(HALF, 1)], row0_v)
        for b in range(BATCH):
            pltpu.sync_copy(row0_v, out_hbm.at[pl.ds(b * MAXLEN, 1)])


def kernel(x, table):
    del x
    return _emb(table).reshape(BATCH, MAXLEN, DM)
